# Initial kernel scaffold; baseline (speedup 1.0000x reference)
import jax
import jax.numpy as jnp
from jax.experimental import pallas as pl


def kernel(node_repr, edge_repr, params, nbr_nodes_d1, nbr_nodes_d2, nbr_nodes_d3, nbr_nodes_d4, nbr_nodes_d5, nbr_edges_d1, nbr_edges_d2, nbr_edges_d3, nbr_edges_d4, nbr_edges_d5):
    raise NotImplementedError("write your pallas kernel here")


# speedup vs baseline: 3.0750x; 3.0750x over previous
import functools

import jax
import jax.numpy as jnp
from jax import lax
from jax.experimental import pallas as pl
from jax.experimental.pallas import tpu as pltpu
from jax.experimental.pallas import tpu_sc as plsc

N = 100000
E = 300000
NS = 128
ES = 16
OUT = 128
DEGS = (1, 2, 3, 4, 5)
GROUP = N // 5
NUM_CONV = 2
EPS = 1e-5
BATCH = 500
MAXLEN = N // BATCH

NC = 2
NSUB = 16
NW = NC * NSUB

CR = {1: 80, 2: 40, 3: 40, 4: 32, 5: 16}
NCH = {d: GROUP // CR[d] for d in DEGS}
CD = {d: CR[d] * d for d in DEGS}
NPW = {d: -(-NCH[d] // NW) for d in DEGS}


def _make_sc_gather_sum(feat: int):
  mesh = plsc.VectorSubcoreMesh(core_axis_name="c", subcore_axis_name="s")
  scratch = (
      [pltpu.VMEM((2, 1, CD[d]), jnp.int32) for d in DEGS]
      + [
          pltpu.VMEM((2, 128, feat), jnp.float32),
          pltpu.VMEM((128, feat), jnp.float32),
          pltpu.SemaphoreType.DMA,
          pltpu.SemaphoreType.DMA,
          pltpu.SemaphoreType.DMA,
          pltpu.SemaphoreType.DMA,
      ]
  )

  @functools.partial(
      pl.kernel,
      out_type=jax.ShapeDtypeStruct((N, feat), jnp.float32),
      mesh=mesh,
      scratch_types=scratch,
  )
  def k(table, i1, i2, i3, i4, i5, out,
        v1, v2, v3, v4, v5, rows, acc, sg0, sg1, si0, si1):
    wid = lax.axis_index("s") * NC + lax.axis_index("c")
    idx_hbm = [i1, i2, i3, i4, i5]
    idx_v = [v1, v2, v3, v4, v5]
    sg = [sg0, sg1]
    si = [si0, si1]

    for di, d in enumerate(DEGS):
      nch, cr, cd = NCH[d], CR[d], CD[d]
      ih, iv = idx_hbm[di], idx_v[di]

      def idx_start(c, b, *, _ih=ih, _iv=iv):
        pltpu.async_copy(_ih.at[c], _iv.at[b], si[b])

      def idx_wait(c, b, *, _ih=ih, _iv=iv):
        pltpu.make_async_copy(_ih.at[c], _iv.at[b], si[b]).wait()

      def g_start(c, b, *, _iv=iv, _cd=cd):
        pltpu.async_copy(
            table.at[_iv.at[b, 0]], rows.at[b, pl.ds(0, _cd)], sg[b])

      def g_wait(c, b, *, _iv=iv, _cd=cd):
        pltpu.make_async_copy(
            table.at[_iv.at[b, 0]], rows.at[b, pl.ds(0, _cd)], sg[b]).wait()

      def consume(c, b, *, _d=d, _cr=cr, _cd=cd, _di=di):
        row0 = _di * GROUP + c * _cr
        if _d == 1:
          pltpu.sync_copy(rows.at[b, pl.ds(0, _cd)], out.at[pl.ds(row0, _cr)])
          return

        @pl.loop(0, _cr)
        def _(r):
          base = r * _d
          for cb in range(feat // 16):
            sl = pl.ds(cb * 16, 16)
            v = rows[b, base, sl]
            for j in range(1, _d):
              v = v + rows[b, base + j, sl]
            acc[r, sl] = v

        pltpu.sync_copy(acc.at[pl.ds(0, _cr)], out.at[pl.ds(row0, _cr)])

      for b in range(2):
        idx_start(wid + b * NW, b)
      for b in range(2):
        idx_wait(wid + b * NW, b)
        g_start(wid + b * NW, b)

      npw = NPW[d]

      @pl.loop(0, npw // 2)
      def _(u):
        for b in range(2):
          t = u * 2 + b
          c = wid + t * NW
          c2 = c + 2 * NW

          @pl.when(c < nch)
          def _():
            g_wait(c, b)

          @pl.when(c2 < nch)
          def _():
            idx_start(c2, b)

          @pl.when(c < nch)
          def _():
            consume(c, b)

          @pl.when(c2 < nch)
          def _():
            idx_wait(c2, b)
            g_start(c2, b)

  return k


_node_gather = _make_sc_gather_sum(NS)


def _reshape_idx(a, d):
  return a.reshape(NCH[d], 1, CD[d])


B1 = 4000
GRID = N // B1
BPG = GROUP // B1


def _k1_body(x_ref, ns_ref, es_ref, wc_ref, wn_ref, we_ref, cb_ref,
             wo_ref, ob_ref, a_ref, atom_ref, s1_ref, s2_ref, *, atom):
  i = pl.program_id(0)
  x = x_ref[...]
  a = jnp.dot(x, wc_ref[...], preferred_element_type=jnp.float32)
  a += jnp.dot(ns_ref[...], wn_ref[0], preferred_element_type=jnp.float32)
  a += jnp.dot(es_ref[...], we_ref[0], preferred_element_type=jnp.float32)
  a += cb_ref[...]
  a_ref[...] = a

  if atom:
    s = jnp.dot(x, wo_ref[...], preferred_element_type=jnp.float32)
    s += ob_ref[...]
    s -= jnp.max(s, axis=1, keepdims=True)
    e = jnp.exp(s)
    atom_ref[...] = e / jnp.sum(e, axis=1, keepdims=True)

  @pl.when(i == 0)
  def _():
    s1_ref[...] = jnp.zeros_like(s1_ref)
    s2_ref[...] = jnp.zeros_like(s2_ref)

  s1_ref[...] += jnp.sum(a, axis=0, keepdims=True)
  s2_ref[...] += jnp.sum(a * a, axis=0, keepdims=True)


def _k1(x, ns, es, wcT, wnT, weT, cb, woT, ob, *, atom):
  row = lambda i: (i, 0)
  fixed = lambda i: (0, 0)
  deg = lambda i: (i // BPG, 0, 0)
  out_shapes = [
      jax.ShapeDtypeStruct((N, OUT), jnp.float32),
      jax.ShapeDtypeStruct((N, OUT), jnp.float32),
      jax.ShapeDtypeStruct((1, OUT), jnp.float32),
      jax.ShapeDtypeStruct((1, OUT), jnp.float32),
  ]
  out_specs = [
      pl.BlockSpec((B1, OUT), row),
      pl.BlockSpec((B1, OUT), row),
      pl.BlockSpec((1, OUT), fixed),
      pl.BlockSpec((1, OUT), fixed),
  ]
  if not atom:
    out_shapes.pop(1)
    out_specs.pop(1)
    body = lambda x_r, ns_r, es_r, wc_r, wn_r, we_r, cb_r, wo_r, ob_r, a_r, s1_r, s2_r: _k1_body(
        x_r, ns_r, es_r, wc_r, wn_r, we_r, cb_r, wo_r, ob_r, a_r, None, s1_r,
        s2_r, atom=False)
  else:
    body = functools.partial(_k1_body, atom=True)
  return pl.pallas_call(
      body,
      grid=(GRID,),
      in_specs=[
          pl.BlockSpec((B1, NS), row),
          pl.BlockSpec((B1, NS), row),
          pl.BlockSpec((B1, ES), row),
          pl.BlockSpec((NS, OUT), fixed),
          pl.BlockSpec((1, NS, OUT), deg),
          pl.BlockSpec((1, ES, OUT), deg),
          pl.BlockSpec((1, OUT), fixed),
          pl.BlockSpec((NS, OUT), fixed),
          pl.BlockSpec((1, OUT), fixed),
      ],
      out_specs=out_specs,
      out_shape=out_shapes,
  )(x, ns, es, wcT, wnT, weT, cb, woT, ob)


def _k2_body(a_ref, s1_ref, s2_ref, atom_ref, wo_ref, ob_ref, x_ref,
             atomo_ref, *, last):
  mean = s1_ref[...] / N
  var = s2_ref[...] / N - mean * mean
  rstd = lax.rsqrt(var + EPS)
  xn = jnp.maximum((a_ref[...] - mean) * rstd, 0.0)
  if not last:
    x_ref[...] = xn
  s = jnp.dot(xn, wo_ref[...], preferred_element_type=jnp.float32)
  s += ob_ref[...]
  s -= jnp.max(s, axis=1, keepdims=True)
  e = jnp.exp(s)
  atomo_ref[...] = atom_ref[...] + e / jnp.sum(e, axis=1, keepdims=True)


def _k2(a, s1, s2, atom_in, woT, ob, *, last):
  row = lambda i: (i, 0)
  fixed = lambda i: (0, 0)
  out_shapes = [
      jax.ShapeDtypeStruct((N, NS), jnp.float32),
      jax.ShapeDtypeStruct((N, OUT), jnp.float32),
  ]
  out_specs = [pl.BlockSpec((B1, NS), row), pl.BlockSpec((B1, OUT), row)]
  if last:
    out_shapes.pop(0)
    out_specs.pop(0)
    body = lambda a_r, s1_r, s2_r, at_r, wo_r, ob_r, ao_r: _k2_body(
        a_r, s1_r, s2_r, at_r, wo_r, ob_r, None, ao_r, last=True)
  else:
    body = functools.partial(_k2_body, last=False)
  return pl.pallas_call(
      body,
      grid=(GRID,),
      in_specs=[
          pl.BlockSpec((B1, OUT), row),
          pl.BlockSpec((1, OUT), fixed),
          pl.BlockSpec((1, OUT), fixed),
          pl.BlockSpec((B1, OUT), row),
          pl.BlockSpec((NS, OUT), fixed),
          pl.BlockSpec((1, OUT), fixed),
      ],
      out_specs=out_specs,
      out_shape=out_shapes,
  )(a, s1, s2, atom_in, woT, ob)


def kernel(node_repr, edge_repr, params,
           nbr_nodes_d1, nbr_nodes_d2, nbr_nodes_d3, nbr_nodes_d4,
           nbr_nodes_d5, nbr_edges_d1, nbr_edges_d2, nbr_edges_d3,
           nbr_edges_d4, nbr_edges_d5):
  nn = {1: nbr_nodes_d1, 2: nbr_nodes_d2, 3: nbr_nodes_d3, 4: nbr_nodes_d4,
        5: nbr_nodes_d5}
  ne = {1: nbr_edges_d1, 2: nbr_edges_d2, 3: nbr_edges_d3, 4: nbr_edges_d4,
        5: nbr_edges_d5}
  nidx = [_reshape_idx(nn[d], d) for d in DEGS]
  eidx = [_reshape_idx(ne[d], d) for d in DEGS]

  woT = [params["outW%d" % l].T for l in range(NUM_CONV + 1)]
  ob = [params["outb%d" % l].reshape(1, OUT) for l in range(NUM_CONV + 1)]
  wcT = [params["convW%d" % l].T for l in range(NUM_CONV)]
  cb = [params["convb%d" % l] for l in range(NUM_CONV)]
  wnT = [jnp.stack([params["degW%d_d%d" % (l, d)][:, :NS].T for d in DEGS])
         for l in range(NUM_CONV)]
  weT = [jnp.stack([params["degW%d_d%d" % (l, d)][:, NS:].T for d in DEGS])
         for l in range(NUM_CONV)]

  x0 = node_repr
  edge_pad = jnp.pad(edge_repr, ((0, 0), (0, NS - ES)))
  es = _node_gather(edge_pad, *eidx)[:, :ES]
  ns0 = _node_gather(x0, *nidx)
  a0, atom0, s1_0, s2_0 = _k1(x0, ns0, es, wcT[0], wnT[0], weT[0], cb[0],
                              woT[0], ob[0], atom=True)
  x1, atom01 = _k2(a0, s1_0, s2_0, atom0, woT[1], ob[1], last=False)
  ns1 = _node_gather(x1, *nidx)
  a1, s1_1, s2_1 = _k1(x1, ns1, es, wcT[1], wnT[1], weT[1], cb[1],
                       woT[1], ob[1], atom=False)
  (atom_final,) = _k2(a1, s1_1, s2_1, atom01, woT[2], ob[2], last=True)
  return atom_final.reshape(BATCH, MAXLEN, OUT)


# speedup vs baseline: 3.2063x; 1.0427x over previous
import functools

import jax
import jax.numpy as jnp
from jax import lax
from jax.experimental import pallas as pl
from jax.experimental.pallas import tpu as pltpu
from jax.experimental.pallas import tpu_sc as plsc

N = 100000
E = 300000
NS = 128
ES = 16
OUT = 128
DEGS = (1, 2, 3, 4, 5)
GROUP = N // 5
NUM_CONV = 2
EPS = 1e-5
BATCH = 500
MAXLEN = N // BATCH

NC = 2
NSUB = 16
NW = NC * NSUB

CR = {1: 80, 2: 40, 3: 40, 4: 32, 5: 16}
NCH = {d: GROUP // CR[d] for d in DEGS}
CD = {d: CR[d] * d for d in DEGS}
NPW = {d: -(-NCH[d] // NW) for d in DEGS}


RING = 4


def _make_sc_gather_sum(feat: int):
  mesh = plsc.VectorSubcoreMesh(core_axis_name="c", subcore_axis_name="s")
  scratch = (
      [pltpu.VMEM((RING, 1, CD[d]), jnp.int32) for d in DEGS]
      + [
          pltpu.VMEM((RING, 128, feat), jnp.float32),
          pltpu.VMEM((2, 128, feat), jnp.float32),
      ]
      + [pltpu.SemaphoreType.DMA] * RING
      + [pltpu.SemaphoreType.DMA] * RING
      + [pltpu.SemaphoreType.DMA] * RING
      + [pltpu.SemaphoreType.DMA] * 2
  )

  @functools.partial(
      pl.kernel,
      out_type=jax.ShapeDtypeStruct((N, feat), jnp.float32),
      mesh=mesh,
      scratch_types=scratch,
  )
  def k(table, i1, i2, i3, i4, i5, out, v1, v2, v3, v4, v5, rows, acc, *sems):
    wid = lax.axis_index("s") * NC + lax.axis_index("c")
    idx_hbm = [i1, i2, i3, i4, i5]
    idx_v = [v1, v2, v3, v4, v5]
    sg = sems[0:RING]
    si = sems[RING:2 * RING]
    so = sems[2 * RING:3 * RING]
    sa = sems[3 * RING:3 * RING + 2]

    for di, d in enumerate(DEGS):
      nch, cr, cd = NCH[d], CR[d], CD[d]
      ih, iv = idx_hbm[di], idx_v[di]

      def idx_start(c, b, *, _ih=ih, _iv=iv):
        pltpu.async_copy(_ih.at[c], _iv.at[b], si[b])

      def idx_wait(c, b, *, _ih=ih, _iv=iv):
        pltpu.make_async_copy(_ih.at[c], _iv.at[b], si[b]).wait()

      def g_start(c, b, *, _iv=iv, _cd=cd):
        pltpu.async_copy(
            table.at[_iv.at[b, 0]], rows.at[b, pl.ds(0, _cd)], sg[b])

      def g_wait(c, b, *, _iv=iv, _cd=cd):
        pltpu.make_async_copy(
            table.at[_iv.at[b, 0]], rows.at[b, pl.ds(0, _cd)], sg[b]).wait()

      def out_start(c, b, a, *, _d=d, _cr=cr, _cd=cd, _di=di):
        row0 = _di * GROUP + c * _cr
        if _d == 1:
          pltpu.async_copy(
              rows.at[b, pl.ds(0, _cd)], out.at[pl.ds(row0, _cr)], so[b])
        else:
          pltpu.async_copy(
              acc.at[a, pl.ds(0, _cr)], out.at[pl.ds(row0, _cr)], sa[a])

      def out_wait(c, b, a, *, _d=d, _cr=cr, _cd=cd, _di=di):
        row0 = _di * GROUP + c * _cr
        if _d == 1:
          pltpu.make_async_copy(
              rows.at[b, pl.ds(0, _cd)], out.at[pl.ds(row0, _cr)],
              so[b]).wait()
        else:
          pltpu.make_async_copy(
              acc.at[a, pl.ds(0, _cr)], out.at[pl.ds(row0, _cr)],
              sa[a]).wait()

      def reduce(b, a, *, _d=d, _cr=cr):
        if _d == 1:
          return

        @pl.loop(0, _cr)
        def _(r):
          base = r * _d
          for cb in range(feat // 16):
            sl = pl.ds(cb * 16, 16)
            v = rows[b, base, sl]
            for j in range(1, _d):
              v = v + rows[b, base + j, sl]
            acc[a, r, sl] = v

      for b in range(RING):
        c = wid + b * NW

        @pl.when(c < nch)
        def _(c=c, b=b):
          idx_start(c, b)

      for b in range(RING):
        c = wid + b * NW

        @pl.when(c < nch)
        def _(c=c, b=b):
          idx_wait(c, b)
          g_start(c, b)

      npw = NPW[d]
      nu = -(-npw // RING)

      @pl.loop(0, nu)
      def _(u):
        for b in range(RING):
          t = u * RING + b
          c = wid + t * NW
          cn = c + RING * NW
          a = b % 2

          @pl.when(c < nch)
          def _(c=c, cn=cn, b=b, a=a, t=t):
            g_wait(c, b)
            @pl.when(cn < nch)
            def _():
              idx_start(cn, b)

            if d != 1:
              @pl.when(t >= 2)
              def _():
                out_wait(c - 2 * NW, b, a)

            reduce(b, a)
            out_start(c, b, a)

          @pl.when(cn < nch)
          def _(c=c, cn=cn, b=b, a=a):
            if d == 1:
              out_wait(c, b, a)
            idx_wait(cn, b)
            g_start(cn, b)

      win = RING if d == 1 else 2
      for q in range(win + 1):
        t_last = npw - 1 - q
        if t_last < 0:
          continue
        c = wid + t_last * NW
        b = t_last % RING
        a = t_last % 2

        @pl.when((c < nch) & (c + win * NW >= nch))
        def _(c=c, b=b, a=a):
          out_wait(c, b, a)

  return k


_node_gather = _make_sc_gather_sum(NS)


def _reshape_idx(a, d):
  return a.reshape(NCH[d], 1, CD[d])


B1 = 4000
GRID = N // B1
BPG = GROUP // B1


def _k1_body(x_ref, ns_ref, es_ref, wc_ref, wn_ref, we_ref, cb_ref,
             wo_ref, ob_ref, a_ref, atom_ref, s1_ref, s2_ref, *, atom):
  i = pl.program_id(0)
  x = x_ref[...]
  a = jnp.dot(x, wc_ref[...], preferred_element_type=jnp.float32)
  a += jnp.dot(ns_ref[...], wn_ref[0], preferred_element_type=jnp.float32)
  a += jnp.dot(es_ref[...], we_ref[0], preferred_element_type=jnp.float32)
  a += cb_ref[...]
  a_ref[...] = a

  if atom:
    s = jnp.dot(x, wo_ref[...], preferred_element_type=jnp.float32)
    s += ob_ref[...]
    s -= jnp.max(s, axis=1, keepdims=True)
    e = jnp.exp(s)
    atom_ref[...] = e / jnp.sum(e, axis=1, keepdims=True)

  @pl.when(i == 0)
  def _():
    s1_ref[...] = jnp.zeros_like(s1_ref)
    s2_ref[...] = jnp.zeros_like(s2_ref)

  s1_ref[...] += jnp.sum(a, axis=0, keepdims=True)
  s2_ref[...] += jnp.sum(a * a, axis=0, keepdims=True)


def _k1(x, ns, es, wcT, wnT, weT, cb, woT, ob, *, atom):
  row = lambda i: (i, 0)
  fixed = lambda i: (0, 0)
  deg = lambda i: (i // BPG, 0, 0)
  out_shapes = [
      jax.ShapeDtypeStruct((N, OUT), jnp.float32),
      jax.ShapeDtypeStruct((N, OUT), jnp.float32),
      jax.ShapeDtypeStruct((1, OUT), jnp.float32),
      jax.ShapeDtypeStruct((1, OUT), jnp.float32),
  ]
  out_specs = [
      pl.BlockSpec((B1, OUT), row),
      pl.BlockSpec((B1, OUT), row),
      pl.BlockSpec((1, OUT), fixed),
      pl.BlockSpec((1, OUT), fixed),
  ]
  if not atom:
    out_shapes.pop(1)
    out_specs.pop(1)
    body = lambda x_r, ns_r, es_r, wc_r, wn_r, we_r, cb_r, wo_r, ob_r, a_r, s1_r, s2_r: _k1_body(
        x_r, ns_r, es_r, wc_r, wn_r, we_r, cb_r, wo_r, ob_r, a_r, None, s1_r,
        s2_r, atom=False)
  else:
    body = functools.partial(_k1_body, atom=True)
  return pl.pallas_call(
      body,
      grid=(GRID,),
      in_specs=[
          pl.BlockSpec((B1, NS), row),
          pl.BlockSpec((B1, NS), row),
          pl.BlockSpec((B1, ES), row),
          pl.BlockSpec((NS, OUT), fixed),
          pl.BlockSpec((1, NS, OUT), deg),
          pl.BlockSpec((1, ES, OUT), deg),
          pl.BlockSpec((1, OUT), fixed),
          pl.BlockSpec((NS, OUT), fixed),
          pl.BlockSpec((1, OUT), fixed),
      ],
      out_specs=out_specs,
      out_shape=out_shapes,
  )(x, ns, es, wcT, wnT, weT, cb, woT, ob)


def _k2_body(a_ref, s1_ref, s2_ref, atom_ref, wo_ref, ob_ref, x_ref,
             atomo_ref, *, last):
  mean = s1_ref[...] / N
  var = s2_ref[...] / N - mean * mean
  rstd = lax.rsqrt(var + EPS)
  xn = jnp.maximum((a_ref[...] - mean) * rstd, 0.0)
  if not last:
    x_ref[...] = xn
  s = jnp.dot(xn, wo_ref[...], preferred_element_type=jnp.float32)
  s += ob_ref[...]
  s -= jnp.max(s, axis=1, keepdims=True)
  e = jnp.exp(s)
  atomo_ref[...] = atom_ref[...] + e / jnp.sum(e, axis=1, keepdims=True)


def _k2(a, s1, s2, atom_in, woT, ob, *, last):
  row = lambda i: (i, 0)
  fixed = lambda i: (0, 0)
  out_shapes = [
      jax.ShapeDtypeStruct((N, NS), jnp.float32),
      jax.ShapeDtypeStruct((N, OUT), jnp.float32),
  ]
  out_specs = [pl.BlockSpec((B1, NS), row), pl.BlockSpec((B1, OUT), row)]
  if last:
    out_shapes.pop(0)
    out_specs.pop(0)
    body = lambda a_r, s1_r, s2_r, at_r, wo_r, ob_r, ao_r: _k2_body(
        a_r, s1_r, s2_r, at_r, wo_r, ob_r, None, ao_r, last=True)
  else:
    body = functools.partial(_k2_body, last=False)
  return pl.pallas_call(
      body,
      grid=(GRID,),
      in_specs=[
          pl.BlockSpec((B1, OUT), row),
          pl.BlockSpec((1, OUT), fixed),
          pl.BlockSpec((1, OUT), fixed),
          pl.BlockSpec((B1, OUT), row),
          pl.BlockSpec((NS, OUT), fixed),
          pl.BlockSpec((1, OUT), fixed),
      ],
      out_specs=out_specs,
      out_shape=out_shapes,
  )(a, s1, s2, atom_in, woT, ob)


def kernel(node_repr, edge_repr, params,
           nbr_nodes_d1, nbr_nodes_d2, nbr_nodes_d3, nbr_nodes_d4,
           nbr_nodes_d5, nbr_edges_d1, nbr_edges_d2, nbr_edges_d3,
           nbr_edges_d4, nbr_edges_d5):
  nn = {1: nbr_nodes_d1, 2: nbr_nodes_d2, 3: nbr_nodes_d3, 4: nbr_nodes_d4,
        5: nbr_nodes_d5}
  ne = {1: nbr_edges_d1, 2: nbr_edges_d2, 3: nbr_edges_d3, 4: nbr_edges_d4,
        5: nbr_edges_d5}
  nidx = [_reshape_idx(nn[d], d) for d in DEGS]
  eidx = [_reshape_idx(ne[d], d) for d in DEGS]

  woT = [params["outW%d" % l].T for l in range(NUM_CONV + 1)]
  ob = [params["outb%d" % l].reshape(1, OUT) for l in range(NUM_CONV + 1)]
  wcT = [params["convW%d" % l].T for l in range(NUM_CONV)]
  cb = [params["convb%d" % l] for l in range(NUM_CONV)]
  wnT = [jnp.stack([params["degW%d_d%d" % (l, d)][:, :NS].T for d in DEGS])
         for l in range(NUM_CONV)]
  weT = [jnp.stack([params["degW%d_d%d" % (l, d)][:, NS:].T for d in DEGS])
         for l in range(NUM_CONV)]

  x0 = node_repr
  edge_pad = jnp.pad(edge_repr, ((0, 0), (0, NS - ES)))
  es = _node_gather(edge_pad, *eidx)[:, :ES]
  ns0 = _node_gather(x0, *nidx)
  a0, atom0, s1_0, s2_0 = _k1(x0, ns0, es, wcT[0], wnT[0], weT[0], cb[0],
                              woT[0], ob[0], atom=True)
  x1, atom01 = _k2(a0, s1_0, s2_0, atom0, woT[1], ob[1], last=False)
  ns1 = _node_gather(x1, *nidx)
  a1, s1_1, s2_1 = _k1(x1, ns1, es, wcT[1], wnT[1], weT[1], cb[1],
                       woT[1], ob[1], atom=False)
  (atom_final,) = _k2(a1, s1_1, s2_1, atom01, woT[2], ob[2], last=True)
  return atom_final.reshape(BATCH, MAXLEN, OUT)
